# baseline (speedup 1.0000x reference)
import functools

import jax
import jax.numpy as jnp
from jax import lax
from jax.experimental import pallas as pl
from jax.experimental.pallas import tpu as pltpu
from jax.experimental.pallas import tpu_sc as plsc

N = 10000
NPAD = 10240
E = 320000
EPAD = 327680
UNITS = EPAD // 128
NC = 2
NS = 16
NW = NC * NS
UW = UNITS // NW
STRIPE = NPAD // NS
G = 128
HROWS = NPAD // 128


def _zero_rows(ref, nrows, d):
    def body(i, carry):
        for j in range(d // 16):
            ref[i, pl.ds(j * 16, 16)] = jnp.zeros((16,), jnp.float32)
        return carry
    lax.fori_loop(0, nrows, body, 0)


DEGW = 128


def _make_sc_deg():
    mesh = plsc.VectorSubcoreMesh(core_axis_name="c", subcore_axis_name="s")

    @functools.partial(
        pl.kernel,
        mesh=mesh,
        out_type=jax.ShapeDtypeStruct((NC, NPAD, DEGW), jnp.float32),
        scratch_types=[
            pltpu.VMEM((UW, 128), jnp.int32),
            pltpu.VMEM((128, DEGW), jnp.float32),
            pltpu.VMEM_SHARED((NPAD, DEGW), jnp.float32),
            pltpu.SemaphoreType.DMA,
        ],
    )
    def deg_kernel(ei_hbm, out_hbm, dstall, ones, acc, sem):
        c = lax.axis_index("c")
        s = lax.axis_index("s")
        wid = s * NC + c
        stripe0 = s * STRIPE
        pltpu.sync_copy(ei_hbm.at[1].at[pl.ds(wid * UW, UW)], dstall)
        _zero_rows(ones, 128, DEGW)
        for k in range(STRIPE // 128):
            pltpu.sync_copy(ones, acc.at[pl.ds(stripe0 + k * 128, 128)])

        def fill(i, carry):
            ones[i, :] = jnp.ones((DEGW,), jnp.float32)
            return carry
        lax.fori_loop(0, 128, fill, 0)
        plsc.subcore_barrier()

        descs = [None] * UW
        for u in range(UW):
            descs[u] = pltpu.async_copy(ones, acc.at[dstall.at[u]], sem,
                                        add=True)
            if u >= 16:
                descs[u - 16].wait()
        for u in range(UW - 16, UW):
            descs[u].wait()
        plsc.subcore_barrier()
        pltpu.sync_copy(acc.at[pl.ds(stripe0, STRIPE)],
                        out_hbm.at[c, pl.ds(stripe0, STRIPE)])

    return deg_kernel


NBUF = 2
CHA = 48
CHB = 16
NT = 16
UA = 144
UB = 16
FASTC = 0


def _make_sc_conv(d):
    mesh = plsc.VectorSubcoreMesh(core_axis_name="c", subcore_axis_name="s")

    @functools.partial(
        pl.kernel,
        mesh=mesh,
        out_type=jax.ShapeDtypeStruct((NC, NPAD, d), jnp.float32),
        scratch_types=[
            pltpu.VMEM((CHA, 128), jnp.int32),
            pltpu.VMEM((CHA, 128), jnp.int32),
            pltpu.VMEM((NBUF, 128, d), jnp.float32),
            pltpu.VMEM_SHARED((NPAD, d), jnp.float32),
            pltpu.SemaphoreType.DMA,
            pltpu.SemaphoreType.DMA,
        ],
    )
    def conv_kernel(ei_hbm, ys_hbm, out_hbm, srcall, dstall, bufs, acc,
                    gsem, ssem):
        c = lax.axis_index("c")
        s = lax.axis_index("s")
        stripe0 = s * STRIPE

        def chunk_pipeline(base, n):
            pltpu.sync_copy(ei_hbm.at[0].at[pl.ds(base, n)],
                            srcall.at[pl.ds(0, n)])
            pltpu.sync_copy(ei_hbm.at[1].at[pl.ds(base, n)],
                            dstall.at[pl.ds(0, n)])
            def gstart(j):
                b = bufs.at[j % NBUF]
                return (
                    pltpu.async_copy(
                        ys_hbm.at[srcall.at[j].at[pl.ds(0, 64)]],
                        b.at[pl.ds(0, 64)], gsem),
                    pltpu.async_copy(
                        ys_hbm.at[srcall.at[j].at[pl.ds(64, 64)]],
                        b.at[pl.ds(64, 64)], gsem),
                )
            gd = [None] * n
            sd = [None] * n
            gd[0] = gstart(0)
            for j in range(n):
                gd[j][0].wait()
                gd[j][1].wait()
                sd[j] = pltpu.async_copy(bufs.at[j % NBUF],
                                         acc.at[dstall.at[j]], ssem,
                                         add=True)
                if j + 1 < n:
                    if j >= 1:
                        sd[j - 1].wait()
                    gd[j + 1] = gstart(j + 1)
            sd[n - 2].wait()
            sd[n - 1].wait()

        _zero_rows(bufs.at[0], 128, d)
        for k in range(STRIPE // 128):
            pltpu.sync_copy(bufs.at[0],
                            acc.at[pl.ds(stripe0 + k * 128, 128)])
        plsc.subcore_barrier()

        @pl.when(c == FASTC)
        def _():
            for ch in range(UA // CHA):
                chunk_pipeline(s * UA + ch * CHA, CHA)

        @pl.when(c != FASTC)
        def _():
            for ch in range(UB // CHB):
                chunk_pipeline(NT * UA + s * UB + ch * CHB, CHB)

        plsc.subcore_barrier()
        pltpu.sync_copy(acc.at[pl.ds(stripe0, STRIPE)],
                        out_hbm.at[c, pl.ds(stripe0, STRIPE)])

    return conv_kernel


def _tc1(dega_ref, degb_ref, x_ref, w1_ref, ys_ref, dinv_ref):
    deg = dega_ref[...] + degb_ref[...] + 1.0
    dinv = lax.rsqrt(deg)
    dinv_ref[...] = dinv
    ys_ref[...] = jnp.dot(x_ref[...], w1_ref[...],
                          preferred_element_type=jnp.float32) * dinv


def _tc2(acc1_ref, ys1_ref, dinv_ref, b1_ref, w34_ref, ys2_ref):
    dinv = dinv_ref[...]
    t = (acc1_ref[0] + acc1_ref[1] + ys1_ref[...]) * dinv + b1_ref[...]
    rows = lax.broadcasted_iota(jnp.int32, (NPAD, 1), 0)
    h = jnp.where(rows < N, jax.nn.relu(t), 0.0)
    ys2_ref[...] = jnp.dot(h, w34_ref[...],
                           preferred_element_type=jnp.float32) * dinv


def _tc3(acc2_ref, ys2_ref, dinv_ref, b34_ref, noise_ref, batch_ref,
         wfc_ref, bfc_ref, out_ref):
    dinv = dinv_ref[...]
    mt = (acc2_ref[0] + acc2_ref[1] + ys2_ref[...]) * dinv + b34_ref[...]
    mean = mt[:, :32]
    log_std = mt[:, 32:64]
    z = mean + noise_ref[...] * jnp.exp(log_std)
    rows = lax.broadcasted_iota(jnp.int32, (NPAD, 1), 0)
    z = jnp.where(rows < N, z, 0.0)
    gids = lax.broadcasted_iota(jnp.int32, (G, NPAD), 0)
    p = (batch_ref[...] == gids).astype(jnp.float32)
    sums = jnp.dot(p, z, preferred_element_type=jnp.float32)
    cnt = jnp.sum(p, axis=1, keepdims=True)
    pooled = sums / jnp.maximum(cnt, 1.0)
    logits = jnp.dot(pooled, wfc_ref[...],
                     preferred_element_type=jnp.float32) + bfc_ref[...]
    mx = jnp.max(logits, axis=1, keepdims=True)
    ex = jnp.exp(logits - mx)
    out_ref[...] = (logits - mx) - jnp.log(jnp.sum(ex, axis=1, keepdims=True))


def kernel(x, edge_index, batch, noise, W1, b1, W3, b3, W4, b4, Wfc, bfc):
    i32 = jnp.int32
    src = edge_index[0]
    dst = edge_index[1]
    padlen = EPAD - E
    src_pad = jnp.concatenate([src, jnp.full((padlen,), N, i32)])
    dst_pad = jnp.concatenate([dst, jnp.full((padlen,), N, i32)])
    ei = jnp.concatenate([src_pad.reshape(1, -1, 128),
                          dst_pad.reshape(1, -1, 128)], axis=0)
    x_pad = jnp.pad(x, ((0, NPAD - N), (0, 0)))
    noise_pad = jnp.pad(noise, ((0, NPAD - N), (0, 0)))
    batch_row = jnp.pad(batch, (0, NPAD - N),
                        constant_values=G).reshape(1, NPAD)
    w34 = jnp.pad(jnp.concatenate([W3, W4], axis=1), ((0, 0), (0, 64)))
    b34 = jnp.pad(jnp.concatenate([b3, b4]), (0, 64)).reshape(1, 128)
    b1r = b1.reshape(1, 128)
    bfcr = bfc.reshape(1, 4)

    deg2 = _make_sc_deg()(ei)
    dega = deg2[0, :, 0:1]
    degb = deg2[1, :, 0:1]

    ys1, dinv = pl.pallas_call(
        _tc1,
        out_shape=(jax.ShapeDtypeStruct((NPAD, 128), jnp.float32),
                   jax.ShapeDtypeStruct((NPAD, 1), jnp.float32)),
    )(dega, degb, x_pad, W1)

    acc1 = _make_sc_conv(128)(ei, ys1)

    ys2 = pl.pallas_call(
        _tc2,
        out_shape=jax.ShapeDtypeStruct((NPAD, 128), jnp.float32),
    )(acc1, ys1, dinv, b1r, w34)

    acc2 = _make_sc_conv(128)(ei, ys2)

    out = pl.pallas_call(
        _tc3,
        out_shape=jax.ShapeDtypeStruct((G, 4), jnp.float32),
    )(acc2, ys2, dinv, b34, noise_pad, batch_row, Wfc, bfcr)

    return out


# speedup vs baseline: 1.0208x; 1.0208x over previous
import functools

import jax
import jax.numpy as jnp
from jax import lax
from jax.experimental import pallas as pl
from jax.experimental.pallas import tpu as pltpu
from jax.experimental.pallas import tpu_sc as plsc

N = 10000
NPAD = 10240
E = 320000
EPAD = 327680
UNITS = EPAD // 128
NC = 2
NS = 16
NW = NC * NS
UW = UNITS // NW
STRIPE = NPAD // NS
G = 128
HROWS = NPAD // 128


def _zero_rows(ref, nrows, d):
    def body(i, carry):
        for j in range(d // 16):
            ref[i, pl.ds(j * 16, 16)] = jnp.zeros((16,), jnp.float32)
        return carry
    lax.fori_loop(0, nrows, body, 0)


DEGW = 128


def _make_sc_deg():
    mesh = plsc.VectorSubcoreMesh(core_axis_name="c", subcore_axis_name="s")

    @functools.partial(
        pl.kernel,
        mesh=mesh,
        out_type=jax.ShapeDtypeStruct((NC, NPAD, DEGW), jnp.float32),
        scratch_types=[
            pltpu.VMEM((UW, 128), jnp.int32),
            pltpu.VMEM((128, DEGW), jnp.float32),
            pltpu.VMEM_SHARED((NPAD, DEGW), jnp.float32),
            pltpu.SemaphoreType.DMA,
        ],
    )
    def deg_kernel(ei_hbm, out_hbm, dstall, ones, acc, sem):
        c = lax.axis_index("c")
        s = lax.axis_index("s")
        wid = s * NC + c
        stripe0 = s * STRIPE
        pltpu.sync_copy(ei_hbm.at[1].at[pl.ds(wid * UW, UW)], dstall)
        _zero_rows(ones, 128, DEGW)
        for k in range(STRIPE // 128):
            pltpu.sync_copy(ones, acc.at[pl.ds(stripe0 + k * 128, 128)])

        def fill(i, carry):
            ones[i, :] = jnp.ones((DEGW,), jnp.float32)
            return carry
        lax.fori_loop(0, 128, fill, 0)
        plsc.subcore_barrier()

        descs = [None] * UW
        for u in range(UW):
            descs[u] = pltpu.async_copy(ones, acc.at[dstall.at[u]], sem,
                                        add=True)
            if u >= 16:
                descs[u - 16].wait()
        for u in range(UW - 16, UW):
            descs[u].wait()
        plsc.subcore_barrier()
        pltpu.sync_copy(acc.at[pl.ds(stripe0, STRIPE)],
                        out_hbm.at[c, pl.ds(stripe0, STRIPE)])

    return deg_kernel


NBUF = 2
CHA = 48
CHB = 16
NT = 16
UA = 144
UB = 16
FASTC = 0


def _make_sc_conv(d):
    mesh = plsc.VectorSubcoreMesh(core_axis_name="c", subcore_axis_name="s")

    @functools.partial(
        pl.kernel,
        mesh=mesh,
        out_type=jax.ShapeDtypeStruct((NC, NPAD, d), jnp.float32),
        scratch_types=[
            pltpu.VMEM((CHA, 128), jnp.int32),
            pltpu.VMEM((CHA, 128), jnp.int32),
            pltpu.VMEM((NBUF, 128, d), jnp.float32),
            pltpu.VMEM_SHARED((NPAD, d), jnp.float32),
            pltpu.SemaphoreType.DMA,
            pltpu.SemaphoreType.DMA,
        ],
    )
    def conv_kernel(ei_hbm, ys_hbm, out_hbm, srcall, dstall, bufs, acc,
                    gsem, ssem):
        c = lax.axis_index("c")
        s = lax.axis_index("s")
        stripe0 = s * STRIPE

        def chunk_pipeline(base, n):
            pltpu.sync_copy(ei_hbm.at[0].at[pl.ds(base, n)],
                            srcall.at[pl.ds(0, n)])
            pltpu.sync_copy(ei_hbm.at[1].at[pl.ds(base, n)],
                            dstall.at[pl.ds(0, n)])
            gd = [None] * n
            sd = [None] * n
            gd[0] = pltpu.async_copy(ys_hbm.at[srcall.at[0]], bufs.at[0],
                                     gsem)
            for j in range(n):
                gd[j].wait()
                sd[j] = pltpu.async_copy(bufs.at[j % NBUF],
                                         acc.at[dstall.at[j]], ssem,
                                         add=True)
                if j + 1 < n:
                    if j >= 1:
                        sd[j - 1].wait()
                    gd[j + 1] = pltpu.async_copy(ys_hbm.at[srcall.at[j + 1]],
                                                 bufs.at[(j + 1) % NBUF],
                                                 gsem)
            sd[n - 2].wait()
            sd[n - 1].wait()

        _zero_rows(bufs.at[0], 128, d)
        for k in range(STRIPE // 128):
            pltpu.sync_copy(bufs.at[0],
                            acc.at[pl.ds(stripe0 + k * 128, 128)])
        plsc.subcore_barrier()

        @pl.when(c == FASTC)
        def _():
            for ch in range(UA // CHA):
                chunk_pipeline(s * UA + ch * CHA, CHA)

        @pl.when(c != FASTC)
        def _():
            for ch in range(UB // CHB):
                chunk_pipeline(NT * UA + s * UB + ch * CHB, CHB)

        plsc.subcore_barrier()
        pltpu.sync_copy(acc.at[pl.ds(stripe0, STRIPE)],
                        out_hbm.at[c, pl.ds(stripe0, STRIPE)])

    return conv_kernel


def _tcmm(x_ref, w1_ref, xw_ref):
    xw_ref[pl.ds(0, N)] = jnp.dot(x_ref[...], w1_ref[...],
                                  preferred_element_type=jnp.float32)
    xw_ref[pl.ds(N, NPAD - N)] = jnp.zeros((NPAD - N, 128), jnp.float32)


def _tc1(deg2_ref, xw_ref, ys_ref, dinv_ref):
    deg = deg2_ref[0, :, 0:1] + deg2_ref[1, :, 0:1] + 1.0
    dinv = lax.rsqrt(deg)
    dinv_ref[...] = dinv
    ys_ref[...] = xw_ref[...] * dinv


def _tc2(acc1_ref, ys1_ref, dinv_ref, b1_ref, w34_ref, ys2_ref):
    dinv = dinv_ref[...]
    t = (acc1_ref[0] + acc1_ref[1] + ys1_ref[...]) * dinv + b1_ref[...]
    rows = lax.broadcasted_iota(jnp.int32, (NPAD, 1), 0)
    h = jnp.where(rows < N, jax.nn.relu(t), 0.0)
    ys2_ref[...] = jnp.dot(h, w34_ref[...],
                           preferred_element_type=jnp.float32) * dinv


def _tc3(acc2_ref, ys2_ref, dinv_ref, b34_ref, noise_ref, batch_ref,
         wfc_ref, bfc_ref, out_ref):
    dinv = dinv_ref[...]
    mt = (acc2_ref[0] + acc2_ref[1] + ys2_ref[...]) * dinv + b34_ref[...]
    mean = mt[:N, :32]
    log_std = mt[:N, 32:64]
    z = mean + noise_ref[...] * jnp.exp(log_std)
    gids = lax.broadcasted_iota(jnp.int32, (G, N), 0)
    p = (batch_ref[...] == gids).astype(jnp.float32)
    sums = jnp.dot(p, z, preferred_element_type=jnp.float32)
    cnt = jnp.sum(p, axis=1, keepdims=True)
    pooled = sums / jnp.maximum(cnt, 1.0)
    logits = jnp.dot(pooled, wfc_ref[...],
                     preferred_element_type=jnp.float32) + bfc_ref[...]
    mx = jnp.max(logits, axis=1, keepdims=True)
    ex = jnp.exp(logits - mx)
    out_ref[...] = (logits - mx) - jnp.log(jnp.sum(ex, axis=1, keepdims=True))


def kernel(x, edge_index, batch, noise, W1, b1, W3, b3, W4, b4, Wfc, bfc):
    i32 = jnp.int32
    padlen = EPAD - E
    ei = jnp.concatenate(
        [edge_index, jnp.full((2, padlen), N, i32)], axis=1
    ).reshape(2, -1, 128)
    batch_row = batch.reshape(1, N)
    w34 = jnp.pad(jnp.concatenate([W3, W4], axis=1), ((0, 0), (0, 64)))
    b34 = jnp.pad(jnp.concatenate([b3, b4]), (0, 64)).reshape(1, 128)
    b1r = b1.reshape(1, 128)
    bfcr = bfc.reshape(1, 4)

    xw1 = pl.pallas_call(
        _tcmm,
        out_shape=jax.ShapeDtypeStruct((NPAD, 128), jnp.float32),
    )(x, W1)

    deg2 = _make_sc_deg()(ei)

    ys1, dinv = pl.pallas_call(
        _tc1,
        out_shape=(jax.ShapeDtypeStruct((NPAD, 128), jnp.float32),
                   jax.ShapeDtypeStruct((NPAD, 1), jnp.float32)),
    )(deg2, xw1)

    acc1 = _make_sc_conv(128)(ei, ys1)

    ys2 = pl.pallas_call(
        _tc2,
        out_shape=jax.ShapeDtypeStruct((NPAD, 128), jnp.float32),
    )(acc1, ys1, dinv, b1r, w34)

    acc2 = _make_sc_conv(128)(ei, ys2)

    out = pl.pallas_call(
        _tc3,
        out_shape=jax.ShapeDtypeStruct((G, 4), jnp.float32),
    )(acc2, ys2, dinv, b34, noise, batch_row, Wfc, bfcr)

    return out
